# baseline (speedup 1.0000x reference)
import functools

import jax
import jax.numpy as jnp
from jax import lax
from jax.experimental import pallas as pl
from jax.experimental.pallas import tpu as pltpu
from jax.experimental.pallas import tpu_sc as plsc

N_NODES = 10000
D = 128
E = 320000
NC, NS = 2, 16
NW = NC * NS
EPW = E // NW
CHUNK = 128
PH, PC = 5, 16
NCHUNK = PH * PC
EPW_PAD = NCHUNK * CHUNK
N_PAD = 10112
RPT = N_PAD // NS
BM = 400


@functools.cache
def _make_sc_scatter():
    mesh = plsc.VectorSubcoreMesh(core_axis_name="c", subcore_axis_name="s",
                                  num_cores=NC, num_subcores=NS)
    scratch = [
        pltpu.VMEM((2, 2, PC, CHUNK), jnp.int32),
        pltpu.VMEM((2, CHUNK, D), jnp.float32),
        pltpu.VMEM_SHARED((N_PAD, D), jnp.float32),
        pltpu.SemaphoreType.DMA,
        pltpu.SemaphoreType.DMA,
        pltpu.SemaphoreType.DMA,
    ]

    def body(y_hbm, edges_hbm, z_hbm, out_hbm,
             idx_v, rows_v, acc_sh, sem_g, sem_s, sem_i):
        c = lax.axis_index("c")
        s = lax.axis_index("s")
        off = pl.multiple_of(s * RPT, 8)
        pltpu.sync_copy(edges_hbm.at[c, s, 0], idx_v.at[0])
        pltpu.async_copy(edges_hbm.at[c, s, 1], idx_v.at[1], sem_i)
        pltpu.sync_copy(z_hbm.at[pl.ds(off, RPT)],
                        acc_sh.at[pl.ds(off, RPT)])
        plsc.subcore_barrier()

        for ph in range(PH):
            b = ph % 2
            if ph > 0:
                pltpu.make_async_copy(edges_hbm.at[c, s, ph],
                                      idx_v.at[b], sem_i).wait()
                if ph + 1 < PH:
                    pltpu.async_copy(edges_hbm.at[c, s, ph + 1],
                                     idx_v.at[1 - b], sem_i)
            sv = idx_v.at[b, 0]
            dv = idx_v.at[b, 1]
            pltpu.async_copy(y_hbm.at[sv.at[0]], rows_v.at[0], sem_g)

            def pair(t, carry):
                j0 = t * 2
                j1 = j0 + 1
                pltpu.make_async_copy(y_hbm.at[sv.at[j0]],
                                      rows_v.at[0], sem_g).wait()
                s0 = pltpu.async_copy(rows_v.at[0], acc_sh.at[dv.at[j0]],
                                      sem_s, add=True)
                pltpu.async_copy(y_hbm.at[sv.at[j1]], rows_v.at[1], sem_g)
                s0.wait()
                pltpu.make_async_copy(y_hbm.at[sv.at[j1]],
                                      rows_v.at[1], sem_g).wait()
                s1 = pltpu.async_copy(rows_v.at[1], acc_sh.at[dv.at[j1]],
                                      sem_s, add=True)

                @pl.when(t < PC // 2 - 1)
                def _():
                    pltpu.async_copy(y_hbm.at[sv.at[j0 + 2]],
                                     rows_v.at[0], sem_g)

                s1.wait()
                return carry

            lax.fori_loop(0, PC // 2, pair, 0)

        plsc.subcore_barrier()
        pltpu.sync_copy(acc_sh.at[pl.ds(off, RPT)],
                        out_hbm.at[c, pl.ds(off, RPT)])

    return pl.kernel(body,
                     out_type=jax.ShapeDtypeStruct((NC, N_PAD, D),
                                                   jnp.float32),
                     mesh=mesh, scratch_types=scratch)


@functools.cache
def _make_sc_deg():
    mesh = plsc.VectorSubcoreMesh(core_axis_name="c", subcore_axis_name="s",
                                  num_cores=NC, num_subcores=NS)
    scratch = [
        pltpu.VMEM((NCHUNK, CHUNK), jnp.int32),
        pltpu.VMEM((CHUNK, D), jnp.float32),
        pltpu.VMEM_SHARED((N_PAD, D), jnp.float32),
    ]

    def body(edges_hbm, z_hbm, ones_hbm, deg_hbm,
             dst_v, ones_v, deg_sh):
        c = lax.axis_index("c")
        s = lax.axis_index("s")
        off = pl.multiple_of(s * RPT, 8)
        for ph in range(PH):
            pltpu.sync_copy(edges_hbm.at[c, s, ph, 1],
                            dst_v.at[pl.ds(ph * PC, PC)])
        pltpu.sync_copy(ones_hbm, ones_v)
        pltpu.sync_copy(z_hbm.at[pl.ds(off, RPT)],
                        deg_sh.at[pl.ds(off, RPT)])
        plsc.subcore_barrier()

        def step(j, carry):
            pltpu.sync_copy(ones_v, deg_sh.at[dst_v.at[j]], add=True)
            return carry

        lax.fori_loop(0, NCHUNK, step, 0)

        plsc.subcore_barrier()
        pltpu.sync_copy(deg_sh.at[pl.ds(off, RPT)],
                        deg_hbm.at[c, pl.ds(off, RPT)])

    return pl.kernel(body,
                     out_type=jax.ShapeDtypeStruct((NC, N_PAD, D),
                                                   jnp.float32),
                     mesh=mesh, scratch_types=scratch)


def _sc_scatter(*args):
    return _make_sc_scatter()(*args)


def _sc_deg(*args):
    return _make_sc_deg()(*args)


def _pack_edges(edge_index):
    ei = edge_index.astype(jnp.int32)
    src = ei[0].reshape(NW, EPW)
    dst = ei[1].reshape(NW, EPW)
    npad = EPW_PAD - EPW
    pad_dst = N_NODES + jnp.arange(npad, dtype=jnp.int32) % (N_PAD - N_NODES)
    src = jnp.concatenate(
        [src, jnp.zeros((NW, npad), jnp.int32)], axis=1)
    dst = jnp.concatenate(
        [dst, jnp.broadcast_to(pad_dst, (NW, npad))], axis=1)
    src = src.reshape(NC, NS, PH, PC, CHUNK)
    dst = dst.reshape(NC, NS, PH, PC, CHUNK)
    return jnp.stack([src, dst], axis=3)


def _matmul2(x, wcat, b):
    n, din = x.shape

    def body(x_ref, w_ref, b_ref, ol_ref, or_ref):
        y = jnp.dot(x_ref[...], w_ref[...],
                    preferred_element_type=jnp.float32)
        ol_ref[...] = y[:, :D]
        or_ref[...] = y[:, D:] + b_ref[0:1]

    return pl.pallas_call(
        body,
        grid=(n // BM,),
        in_specs=[pl.BlockSpec((BM, din), lambda i: (i, 0)),
                  pl.BlockSpec((din, 2 * D), lambda i: (0, 0)),
                  pl.BlockSpec((8, D), lambda i: (0, 0))],
        out_specs=[pl.BlockSpec((BM, D), lambda i: (i, 0)),
                   pl.BlockSpec((BM, D), lambda i: (i, 0))],
        out_shape=[jax.ShapeDtypeStruct((n, D), jnp.float32)] * 2,
    )(x, wcat, b)


def _mid(acc, deg, y_r, wcat, b):
    def body(acc_ref, deg_ref, yr_ref, w_ref, b_ref, ol_ref, or_ref):
        a = acc_ref[0] + acc_ref[1]
        dg = deg_ref[0][:, :1] + deg_ref[1][:, :1]
        inv = 1.0 / jnp.maximum(dg, 1.0)
        h = jnp.maximum(a * inv + yr_ref[...], 0.0)
        y = jnp.dot(h, w_ref[...], preferred_element_type=jnp.float32)
        ol_ref[...] = y[:, :D]
        or_ref[...] = y[:, D:] + b_ref[0:1]

    return pl.pallas_call(
        body,
        grid=(N_NODES // BM,),
        in_specs=[pl.BlockSpec((NC, BM, D), lambda i: (0, i, 0)),
                  pl.BlockSpec((NC, BM, D), lambda i: (0, i, 0)),
                  pl.BlockSpec((BM, D), lambda i: (i, 0)),
                  pl.BlockSpec((D, 2 * D), lambda i: (0, 0)),
                  pl.BlockSpec((8, D), lambda i: (0, 0))],
        out_specs=[pl.BlockSpec((BM, D), lambda i: (i, 0)),
                   pl.BlockSpec((BM, D), lambda i: (i, 0))],
        out_shape=[jax.ShapeDtypeStruct((N_NODES, D), jnp.float32)] * 2,
    )(acc, deg, y_r, wcat, b)


def _final(acc, deg, y_r):
    def body(acc_ref, deg_ref, yr_ref, o_ref):
        a = acc_ref[0] + acc_ref[1]
        dg = deg_ref[0][:, :1] + deg_ref[1][:, :1]
        inv = 1.0 / jnp.maximum(dg, 1.0)
        o_ref[...] = a * inv + yr_ref[...]

    return pl.pallas_call(
        body,
        grid=(N_NODES // BM,),
        in_specs=[pl.BlockSpec((NC, BM, D), lambda i: (0, i, 0)),
                  pl.BlockSpec((NC, BM, D), lambda i: (0, i, 0)),
                  pl.BlockSpec((BM, D), lambda i: (i, 0))],
        out_specs=pl.BlockSpec((BM, D), lambda i: (i, 0)),
        out_shape=jax.ShapeDtypeStruct((N_NODES, D), jnp.float32),
    )(acc, deg, y_r)


def kernel(x, edge_index, W_l1, b_l1, W_r1, W_l2, b_l2, W_r2):
    edges = _pack_edges(edge_index)
    z = jnp.zeros((N_PAD, D), jnp.float32)
    ones = jnp.ones((CHUNK, D), jnp.float32)
    b1 = jnp.broadcast_to(b_l1, (8, D))
    b2 = jnp.broadcast_to(b_l2, (8, D))

    y1l, y1r = _matmul2(x, jnp.concatenate([W_l1, W_r1], axis=1), b1)
    deg = _sc_deg(edges, z, ones)
    acc1 = _sc_scatter(y1l, edges, z)
    y2l, y2r = _mid(acc1, deg, y1r, jnp.concatenate([W_l2, W_r2], axis=1), b2)
    acc2 = _sc_scatter(y2l, edges, z)
    return _final(acc2, deg, y2r)


# speedup vs baseline: 1.0090x; 1.0090x over previous
import functools

import jax
import jax.numpy as jnp
from jax import lax
from jax.experimental import pallas as pl
from jax.experimental.pallas import tpu as pltpu
from jax.experimental.pallas import tpu_sc as plsc

N_NODES = 10000
D = 128
E = 320000
NC, NS = 2, 16
NW = NC * NS
EPW = E // NW
CHUNK = 128
NCHUNK = 80
EPW_PAD = NCHUNK * CHUNK
N_PAD = 10112
RPT = N_PAD // NS
BM = 400


@functools.cache
def _make_sc_scatter():
    mesh = plsc.VectorSubcoreMesh(core_axis_name="c", subcore_axis_name="s",
                                  num_cores=NC, num_subcores=NS)
    scratch = [
        pltpu.VMEM((NCHUNK, CHUNK), jnp.int32),
        pltpu.VMEM((8, CHUNK), jnp.int32),
        pltpu.VMEM((2, CHUNK, D), jnp.float32),
        pltpu.VMEM_SHARED((N_PAD, D), jnp.float32),
        pltpu.SemaphoreType.DMA,
        pltpu.SemaphoreType.DMA,
    ]

    def body(y_hbm, edges_hbm, z_hbm, out_hbm,
             pk_v, idx32, rows_v, acc_sh, sem_g, sem_s):
        c = lax.axis_index("c")
        s = lax.axis_index("s")
        off = pl.multiple_of(s * RPT, 8)
        pltpu.sync_copy(edges_hbm.at[c, s], pk_v)
        pltpu.sync_copy(z_hbm.at[pl.ds(off, RPT)],
                        acc_sh.at[pl.ds(off, RPT)])
        plsc.subcore_barrier()

        def unpack(j, slot):
            for k in range(CHUNK // 16):
                v = pk_v[j, pl.ds(k * 16, 16)]
                idx32[2 * slot, pl.ds(k * 16, 16)] = v & 0xFFFF
                idx32[2 * slot + 1, pl.ds(k * 16, 16)] = (
                    lax.shift_right_logical(v, 16))

        unpack(0, 0)
        pltpu.async_copy(y_hbm.at[idx32.at[0]], rows_v.at[0], sem_g)

        def pair(t, carry):
            j0 = t * 2
            unpack(j0 + 1, 1)
            pltpu.make_async_copy(y_hbm.at[idx32.at[0]],
                                  rows_v.at[0], sem_g).wait()
            s0 = pltpu.async_copy(rows_v.at[0], acc_sh.at[idx32.at[1]],
                                  sem_s, add=True)
            pltpu.async_copy(y_hbm.at[idx32.at[2]], rows_v.at[1], sem_g)
            s0.wait()

            @pl.when(t < NCHUNK // 2 - 1)
            def _():
                unpack(j0 + 2, 0)

            pltpu.make_async_copy(y_hbm.at[idx32.at[2]],
                                  rows_v.at[1], sem_g).wait()
            s1 = pltpu.async_copy(rows_v.at[1], acc_sh.at[idx32.at[3]],
                                  sem_s, add=True)

            @pl.when(t < NCHUNK // 2 - 1)
            def _():
                pltpu.async_copy(y_hbm.at[idx32.at[0]],
                                 rows_v.at[0], sem_g)

            s1.wait()
            return carry

        lax.fori_loop(0, NCHUNK // 2, pair, 0)

        plsc.subcore_barrier()
        pltpu.sync_copy(acc_sh.at[pl.ds(off, RPT)],
                        out_hbm.at[c, pl.ds(off, RPT)])

    return pl.kernel(body,
                     out_type=jax.ShapeDtypeStruct((NC, N_PAD, D),
                                                   jnp.float32),
                     mesh=mesh, scratch_types=scratch)


@functools.cache
def _make_sc_deg():
    mesh = plsc.VectorSubcoreMesh(core_axis_name="c", subcore_axis_name="s",
                                  num_cores=NC, num_subcores=NS)
    scratch = [
        pltpu.VMEM((NCHUNK, CHUNK), jnp.int32),
        pltpu.VMEM((CHUNK, D), jnp.float32),
        pltpu.VMEM_SHARED((N_PAD, D), jnp.float32),
    ]

    def body(dst32_hbm, z_hbm, ones_hbm, deg_hbm,
             dst_v, ones_v, deg_sh):
        c = lax.axis_index("c")
        s = lax.axis_index("s")
        off = pl.multiple_of(s * RPT, 8)
        pltpu.sync_copy(dst32_hbm.at[c, s], dst_v)
        pltpu.sync_copy(ones_hbm, ones_v)
        pltpu.sync_copy(z_hbm.at[pl.ds(off, RPT)],
                        deg_sh.at[pl.ds(off, RPT)])
        plsc.subcore_barrier()

        def step(j, carry):
            pltpu.sync_copy(ones_v, deg_sh.at[dst_v.at[j]], add=True)
            return carry

        lax.fori_loop(0, NCHUNK, step, 0)

        plsc.subcore_barrier()
        pltpu.sync_copy(deg_sh.at[pl.ds(off, RPT)],
                        deg_hbm.at[c, pl.ds(off, RPT)])

    return pl.kernel(body,
                     out_type=jax.ShapeDtypeStruct((NC, N_PAD, D),
                                                   jnp.float32),
                     mesh=mesh, scratch_types=scratch)


def _sc_scatter(*args):
    return _make_sc_scatter()(*args)


def _sc_deg(*args):
    return _make_sc_deg()(*args)


def _pack_edges(edge_index):
    ei = edge_index.astype(jnp.int32)
    src = ei[0].reshape(NW, EPW)
    dst = ei[1].reshape(NW, EPW)
    npad = EPW_PAD - EPW
    pad_dst = N_NODES + jnp.arange(npad, dtype=jnp.int32) % (N_PAD - N_NODES)
    src = jnp.concatenate(
        [src, jnp.zeros((NW, npad), jnp.int32)], axis=1)
    dst = jnp.concatenate(
        [dst, jnp.broadcast_to(pad_dst, (NW, npad))], axis=1)
    src = src.reshape(NC, NS, NCHUNK, CHUNK)
    dst = dst.reshape(NC, NS, NCHUNK, CHUNK)
    packed = src | (dst << 16)
    return packed, dst


def _matmul2(x, wcat, b):
    n, din = x.shape

    def body(x_ref, w_ref, b_ref, ol_ref, or_ref):
        y = jnp.dot(x_ref[...], w_ref[...],
                    preferred_element_type=jnp.float32)
        ol_ref[...] = y[:, :D]
        or_ref[...] = y[:, D:] + b_ref[0:1]

    return pl.pallas_call(
        body,
        grid=(n // BM,),
        in_specs=[pl.BlockSpec((BM, din), lambda i: (i, 0)),
                  pl.BlockSpec((din, 2 * D), lambda i: (0, 0)),
                  pl.BlockSpec((8, D), lambda i: (0, 0))],
        out_specs=[pl.BlockSpec((BM, D), lambda i: (i, 0)),
                   pl.BlockSpec((BM, D), lambda i: (i, 0))],
        out_shape=[jax.ShapeDtypeStruct((n, D), jnp.float32)] * 2,
    )(x, wcat, b)


def _mid(acc, deg, y_r, wcat, b):
    def body(acc_ref, deg_ref, yr_ref, w_ref, b_ref, ol_ref, or_ref):
        a = acc_ref[0] + acc_ref[1]
        dg = deg_ref[0][:, :1] + deg_ref[1][:, :1]
        inv = 1.0 / jnp.maximum(dg, 1.0)
        h = jnp.maximum(a * inv + yr_ref[...], 0.0)
        y = jnp.dot(h, w_ref[...], preferred_element_type=jnp.float32)
        ol_ref[...] = y[:, :D]
        or_ref[...] = y[:, D:] + b_ref[0:1]

    return pl.pallas_call(
        body,
        grid=(N_NODES // BM,),
        in_specs=[pl.BlockSpec((NC, BM, D), lambda i: (0, i, 0)),
                  pl.BlockSpec((NC, BM, D), lambda i: (0, i, 0)),
                  pl.BlockSpec((BM, D), lambda i: (i, 0)),
                  pl.BlockSpec((D, 2 * D), lambda i: (0, 0)),
                  pl.BlockSpec((8, D), lambda i: (0, 0))],
        out_specs=[pl.BlockSpec((BM, D), lambda i: (i, 0)),
                   pl.BlockSpec((BM, D), lambda i: (i, 0))],
        out_shape=[jax.ShapeDtypeStruct((N_NODES, D), jnp.float32)] * 2,
    )(acc, deg, y_r, wcat, b)


def _final(acc, deg, y_r):
    def body(acc_ref, deg_ref, yr_ref, o_ref):
        a = acc_ref[0] + acc_ref[1]
        dg = deg_ref[0][:, :1] + deg_ref[1][:, :1]
        inv = 1.0 / jnp.maximum(dg, 1.0)
        o_ref[...] = a * inv + yr_ref[...]

    return pl.pallas_call(
        body,
        grid=(N_NODES // BM,),
        in_specs=[pl.BlockSpec((NC, BM, D), lambda i: (0, i, 0)),
                  pl.BlockSpec((NC, BM, D), lambda i: (0, i, 0)),
                  pl.BlockSpec((BM, D), lambda i: (i, 0))],
        out_specs=pl.BlockSpec((BM, D), lambda i: (i, 0)),
        out_shape=jax.ShapeDtypeStruct((N_NODES, D), jnp.float32),
    )(acc, deg, y_r)


def kernel(x, edge_index, W_l1, b_l1, W_r1, W_l2, b_l2, W_r2):
    edges, dst32 = _pack_edges(edge_index)
    z = jnp.zeros((N_PAD, D), jnp.float32)
    ones = jnp.ones((CHUNK, D), jnp.float32)
    b1 = jnp.broadcast_to(b_l1, (8, D))
    b2 = jnp.broadcast_to(b_l2, (8, D))

    y1l, y1r = _matmul2(x, jnp.concatenate([W_l1, W_r1], axis=1), b1)
    deg = _sc_deg(dst32, z, ones)
    acc1 = _sc_scatter(y1l, edges, z)
    y2l, y2r = _mid(acc1, deg, y1r, jnp.concatenate([W_l2, W_r2], axis=1), b2)
    acc2 = _sc_scatter(y2l, edges, z)
    return _final(acc2, deg, y2r)


# speedup vs baseline: 1.2985x; 1.2869x over previous
import functools

import jax
import jax.numpy as jnp
from jax import lax
from jax.experimental import pallas as pl
from jax.experimental.pallas import tpu as pltpu
from jax.experimental.pallas import tpu_sc as plsc

N_NODES = 10000
D = 128
E = 320000
NC, NS = 2, 16
NW = NC * NS
EPW = E // NW
CHUNK = 128
NCHUNK = -(-EPW // CHUNK)
EPW_PAD = NCHUNK * CHUNK
N_PAD = 10112
RPT = N_PAD // NS
BM = 400


@functools.cache
def _make_sc_scatter():
    mesh = plsc.VectorSubcoreMesh(core_axis_name="c", subcore_axis_name="s",
                                  num_cores=NC, num_subcores=NS)
    scratch = [
        pltpu.VMEM((NCHUNK, CHUNK), jnp.int32),
        pltpu.VMEM((NCHUNK, CHUNK), jnp.int32),
        pltpu.VMEM((CHUNK, D), jnp.float32),
        pltpu.VMEM_SHARED((N_PAD, D), jnp.float32),
        pltpu.SemaphoreType.DMA,
    ]

    def body(y_hbm, edges_hbm, z_hbm, out_hbm,
             src_v, dst_v, rows_v, acc_sh, sem_g):
        c = lax.axis_index("c")
        s = lax.axis_index("s")
        off = pl.multiple_of(s * RPT, 8)
        pltpu.sync_copy(edges_hbm.at[c, s, 0], src_v)
        pltpu.sync_copy(edges_hbm.at[c, s, 1], dst_v)
        pltpu.sync_copy(z_hbm.at[pl.ds(off, RPT)],
                        acc_sh.at[pl.ds(off, RPT)])
        plsc.subcore_barrier()

        def step(j, carry):
            pltpu.async_copy(y_hbm.at[src_v.at[j]], rows_v, sem_g).wait()
            pltpu.sync_copy(rows_v, acc_sh.at[dst_v.at[j]], add=True)
            return carry

        lax.fori_loop(0, NCHUNK, step, 0)

        plsc.subcore_barrier()
        pltpu.sync_copy(acc_sh.at[pl.ds(off, RPT)],
                        out_hbm.at[c, pl.ds(off, RPT)])

    return pl.kernel(body,
                     out_type=jax.ShapeDtypeStruct((NC, N_PAD, D),
                                                   jnp.float32),
                     mesh=mesh, scratch_types=scratch)


@functools.cache
def _make_sc_deg():
    mesh = plsc.VectorSubcoreMesh(core_axis_name="c", subcore_axis_name="s",
                                  num_cores=NC, num_subcores=NS)
    scratch = [
        pltpu.VMEM((NCHUNK, CHUNK), jnp.int32),
        pltpu.VMEM((CHUNK, D), jnp.float32),
        pltpu.VMEM_SHARED((N_PAD, D), jnp.float32),
    ]

    def body(edges_hbm, z_hbm, ones_hbm, deg_hbm,
             dst_v, ones_v, deg_sh):
        c = lax.axis_index("c")
        s = lax.axis_index("s")
        off = pl.multiple_of(s * RPT, 8)
        pltpu.sync_copy(edges_hbm.at[c, s, 1], dst_v)
        pltpu.sync_copy(ones_hbm, ones_v)
        pltpu.sync_copy(z_hbm.at[pl.ds(off, RPT)],
                        deg_sh.at[pl.ds(off, RPT)])
        plsc.subcore_barrier()

        def step(j, carry):
            pltpu.sync_copy(ones_v, deg_sh.at[dst_v.at[j]], add=True)
            return carry

        lax.fori_loop(0, NCHUNK, step, 0)

        plsc.subcore_barrier()
        pltpu.sync_copy(deg_sh.at[pl.ds(off, RPT)],
                        deg_hbm.at[c, pl.ds(off, RPT)])

    return pl.kernel(body,
                     out_type=jax.ShapeDtypeStruct((NC, N_PAD, D),
                                                   jnp.float32),
                     mesh=mesh, scratch_types=scratch)


def _sc_scatter(*args):
    return _make_sc_scatter()(*args)


def _sc_deg(*args):
    return _make_sc_deg()(*args)


def _pack_edges(edge_index):
    ei = edge_index.astype(jnp.int32)
    src = ei[0].reshape(NW, EPW)
    dst = ei[1].reshape(NW, EPW)
    npad = EPW_PAD - EPW
    pad_dst = N_NODES + jnp.arange(npad, dtype=jnp.int32) % (N_PAD - N_NODES)
    src = jnp.concatenate(
        [src, jnp.zeros((NW, npad), jnp.int32)], axis=1)
    dst = jnp.concatenate(
        [dst, jnp.broadcast_to(pad_dst, (NW, npad))], axis=1)
    src = src.reshape(NC, NS, NCHUNK, CHUNK)
    dst = dst.reshape(NC, NS, NCHUNK, CHUNK)
    return jnp.stack([src, dst], axis=2)


def _matmul2(x, wcat, b):
    n, din = x.shape

    def body(x_ref, w_ref, b_ref, ol_ref, or_ref):
        y = jnp.dot(x_ref[...], w_ref[...],
                    preferred_element_type=jnp.float32)
        ol_ref[...] = y[:, :D]
        or_ref[...] = y[:, D:] + b_ref[0:1]

    return pl.pallas_call(
        body,
        grid=(n // BM,),
        in_specs=[pl.BlockSpec((BM, din), lambda i: (i, 0)),
                  pl.BlockSpec((din, 2 * D), lambda i: (0, 0)),
                  pl.BlockSpec((8, D), lambda i: (0, 0))],
        out_specs=[pl.BlockSpec((BM, D), lambda i: (i, 0)),
                   pl.BlockSpec((BM, D), lambda i: (i, 0))],
        out_shape=[jax.ShapeDtypeStruct((n, D), jnp.float32)] * 2,
    )(x, wcat, b)


def _mid(acc, deg, y_r, wcat, b):
    def body(acc_ref, deg_ref, yr_ref, w_ref, b_ref, ol_ref, or_ref):
        a = acc_ref[0] + acc_ref[1]
        dg = deg_ref[0][:, :1] + deg_ref[1][:, :1]
        inv = 1.0 / jnp.maximum(dg, 1.0)
        h = jnp.maximum(a * inv + yr_ref[...], 0.0)
        y = jnp.dot(h, w_ref[...], preferred_element_type=jnp.float32)
        ol_ref[...] = y[:, :D]
        or_ref[...] = y[:, D:] + b_ref[0:1]

    return pl.pallas_call(
        body,
        grid=(N_NODES // BM,),
        in_specs=[pl.BlockSpec((NC, BM, D), lambda i: (0, i, 0)),
                  pl.BlockSpec((NC, BM, D), lambda i: (0, i, 0)),
                  pl.BlockSpec((BM, D), lambda i: (i, 0)),
                  pl.BlockSpec((D, 2 * D), lambda i: (0, 0)),
                  pl.BlockSpec((8, D), lambda i: (0, 0))],
        out_specs=[pl.BlockSpec((BM, D), lambda i: (i, 0)),
                   pl.BlockSpec((BM, D), lambda i: (i, 0))],
        out_shape=[jax.ShapeDtypeStruct((N_NODES, D), jnp.float32)] * 2,
    )(acc, deg, y_r, wcat, b)


def _final(acc, deg, y_r):
    def body(acc_ref, deg_ref, yr_ref, o_ref):
        a = acc_ref[0] + acc_ref[1]
        dg = deg_ref[0][:, :1] + deg_ref[1][:, :1]
        inv = 1.0 / jnp.maximum(dg, 1.0)
        o_ref[...] = a * inv + yr_ref[...]

    return pl.pallas_call(
        body,
        grid=(N_NODES // BM,),
        in_specs=[pl.BlockSpec((NC, BM, D), lambda i: (0, i, 0)),
                  pl.BlockSpec((NC, BM, D), lambda i: (0, i, 0)),
                  pl.BlockSpec((BM, D), lambda i: (i, 0))],
        out_specs=pl.BlockSpec((BM, D), lambda i: (i, 0)),
        out_shape=jax.ShapeDtypeStruct((N_NODES, D), jnp.float32),
    )(acc, deg, y_r)


def kernel(x, edge_index, W_l1, b_l1, W_r1, W_l2, b_l2, W_r2):
    edges = _pack_edges(edge_index)
    z = jnp.zeros((N_PAD, D), jnp.float32)
    ones = jnp.ones((CHUNK, D), jnp.float32)
    b1 = jnp.broadcast_to(b_l1, (8, D))
    b2 = jnp.broadcast_to(b_l2, (8, D))

    y1l, y1r = _matmul2(x, jnp.concatenate([W_l1, W_r1], axis=1), b1)
    deg = _sc_deg(edges, z, ones)
    acc1 = _sc_scatter(y1l, edges, z)
    y2l, y2r = _mid(acc1, deg, y1r, jnp.concatenate([W_l2, W_r2], axis=1), b2)
    acc2 = _sc_scatter(y2l, edges, z)
    return _final(acc2, deg, y2r)
